# baseline (speedup 1.0000x reference)
import functools

import jax
import jax.numpy as jnp
from jax.experimental import pallas as pl
from jax.experimental.pallas import tpu as pltpu


def _fused_body(n_experts, eidx_ref, x_ref, wlow_hbm, uin_ref, uout_hbm,
                out_ref, w_ref, uout_ref, sem_w, sem_u):
    i = pl.program_id(0)

    @pl.when(i == 0)
    def _start():
        pltpu.make_async_copy(wlow_hbm, w_ref, sem_w).start()
        pltpu.make_async_copy(uout_hbm, uout_ref, sem_u).start()

    h = jax.lax.dot_general(
        x_ref[...], uin_ref[...], (((1,), (0,)), ((), ())),
        preferred_element_type=jnp.float32,
    ).astype(jnp.bfloat16)

    @pl.when(i == 0)
    def _wait():
        pltpu.make_async_copy(wlow_hbm, w_ref, sem_w).wait()
        pltpu.make_async_copy(uout_hbm, uout_ref, sem_u).wait()

    eidx = eidx_ref[0]
    acc = None
    for e in range(n_experts):
        z_e = jax.lax.dot_general(
            h, w_ref[e], (((1,), (1,)), ((), ())),
            preferred_element_type=jnp.float32,
        )
        acc = z_e if acc is None else jnp.where(eidx == e, z_e, acc)
    z = acc.astype(jnp.bfloat16)
    out_ref[...] = jax.lax.dot_general(
        z, uout_ref[...], (((1,), (1,)), ((), ())),
        preferred_element_type=jnp.float32,
    ).astype(jnp.bfloat16)


@jax.jit
def kernel(x, expert_indices, W_low, U_in, U_out):
    t, d_model = x.shape
    n_experts, u, d = W_low.shape
    tb = 512
    nb = t // tb
    eidx3 = expert_indices.astype(jnp.int32).reshape(nb, tb, 1)
    return pl.pallas_call(
        functools.partial(_fused_body, n_experts),
        grid=(nb,),
        in_specs=[
            pl.BlockSpec((1, tb, 1), lambda i: (i, 0, 0)),
            pl.BlockSpec((tb, d_model), lambda i: (i, 0)),
            pl.BlockSpec(memory_space=pl.ANY),
            pl.BlockSpec((d_model, d), lambda i: (0, 0)),
            pl.BlockSpec(memory_space=pl.ANY),
        ],
        out_specs=pl.BlockSpec((tb, d_model), lambda i: (i, 0)),
        out_shape=jax.ShapeDtypeStruct((t, d_model), jnp.bfloat16),
        scratch_shapes=[
            pltpu.VMEM((n_experts, u, d), jnp.bfloat16),
            pltpu.VMEM((d_model, u), jnp.bfloat16),
            pltpu.SemaphoreType.DMA,
            pltpu.SemaphoreType.DMA,
        ],
        compiler_params=pltpu.CompilerParams(
            dimension_semantics=("arbitrary",),
        ),
    )(eidx3, x, W_low, U_in, U_out)


# speedup vs baseline: 1.0201x; 1.0201x over previous
import functools

import jax
import jax.numpy as jnp
from jax.experimental import pallas as pl
from jax.experimental.pallas import tpu as pltpu


def _fused_body(n_experts, eidx_ref, x_ref, w_ref, uin_ref, uout_ref, out_ref):
    h = jax.lax.dot_general(
        x_ref[...], uin_ref[...], (((1,), (0,)), ((), ())),
        preferred_element_type=jnp.float32,
    ).astype(jnp.bfloat16)
    eidx = eidx_ref[0]
    acc = None
    for e in range(n_experts):
        z_e = jax.lax.dot_general(
            h, w_ref[e], (((1,), (1,)), ((), ())),
            preferred_element_type=jnp.float32,
        )
        acc = z_e if acc is None else jnp.where(eidx == e, z_e, acc)
    z = acc.astype(jnp.bfloat16)
    out_ref[...] = jax.lax.dot_general(
        z, uout_ref[...], (((1,), (1,)), ((), ())),
        preferred_element_type=jnp.float32,
    ).astype(jnp.bfloat16)


@jax.jit
def kernel(x, expert_indices, W_low, U_in, U_out):
    t, d_model = x.shape
    n_experts, u, d = W_low.shape
    tb = 512
    nb = t // tb
    eidx3 = expert_indices.astype(jnp.int32).reshape(nb, tb, 1)
    return pl.pallas_call(
        functools.partial(_fused_body, n_experts),
        grid=(nb,),
        in_specs=[
            pl.BlockSpec((1, tb, 1), lambda i: (i, 0, 0)),
            pl.BlockSpec((tb, d_model), lambda i: (i, 0)),
            pl.BlockSpec((n_experts, u, d), lambda i: (0, 0, 0)),
            pl.BlockSpec((d_model, d), lambda i: (0, 0)),
            pl.BlockSpec((d_model, u), lambda i: (0, 0)),
        ],
        out_specs=pl.BlockSpec((tb, d_model), lambda i: (i, 0)),
        out_shape=jax.ShapeDtypeStruct((t, d_model), jnp.bfloat16),
        compiler_params=pltpu.CompilerParams(
            dimension_semantics=("parallel",),
        ),
    )(eidx3, x, W_low, U_in, U_out)
